# baseline (speedup 1.0000x reference)
import functools

import jax
import jax.numpy as jnp
from jax import lax
from jax.experimental import pallas as pl
from jax.experimental.pallas import tpu as pltpu
from jax.experimental.pallas import tpu_sc as plsc

N = 10000
DEG = 32
D = 128
OUT = 128

_NC = 2
_NS = 16
_NW = _NC * _NS

_CN = 8
_CROWS = _CN * DEG
_NCHUNK = N // _CN
_CPW = -(-_NCHUNK // _NW)

_LANE = 16
_JV = D // _LANE


def _sc_sum_body(h2_hbm, out_hbm, buf, sums):
    wid = lax.axis_index("s") * _NC + lax.axis_index("c")

    def chunk_body(i, carry):
        cid = wid + i * _NW

        @pl.when(cid < _NCHUNK)
        def _():
            row0 = cid * _CROWS
            pltpu.sync_copy(h2_hbm.at[pl.ds(row0, _CROWS)], buf)
            for n in range(_CN):
                accs = tuple(jnp.zeros((_LANE,), jnp.float32) for _ in range(_JV))

                def deg_body(g, accs, n=n):
                    base = n * DEG + g * 8
                    for dd in range(8):
                        accs = tuple(
                            accs[j] + buf[base + dd, pl.ds(j * _LANE, _LANE)]
                            for j in range(_JV)
                        )
                    return accs

                accs = lax.fori_loop(0, DEG // 8, deg_body, accs)
                for j in range(_JV):
                    sums[n, pl.ds(j * _LANE, _LANE)] = accs[j]
            pltpu.sync_copy(sums, out_hbm.at[pl.ds(cid * _CN, _CN)])

        return carry

    lax.fori_loop(0, _CPW, chunk_body, 0)


@functools.partial(jax.jit, donate_argnums=())
def _sc_mailbox_sum(h2):
    mesh = plsc.VectorSubcoreMesh(
        core_axis_name="c", subcore_axis_name="s",
        num_cores=_NC, num_subcores=_NS,
    )
    return pl.kernel(
        _sc_sum_body,
        out_type=jax.ShapeDtypeStruct((N, D), jnp.float32),
        mesh=mesh,
        scratch_types=[
            pltpu.VMEM((_CROWS, D), jnp.float32),
            pltpu.VMEM((_CN, D), jnp.float32),
        ],
    )(h2)


_BLK = 2000


def _tc_linear_body(sums_ref, nf_ref, w1t_ref, w2t_ref, b_ref, out_ref):
    out_ref[...] = (
        jnp.dot(sums_ref[...], w1t_ref[...], preferred_element_type=jnp.float32)
        + jnp.dot(nf_ref[...], w2t_ref[...], preferred_element_type=jnp.float32)
        + b_ref[...]
    )


def kernel(h, node_feat, W, b):
    h2 = h.reshape(N * DEG, D)
    sums = _sc_mailbox_sum(h2)
    w1t = W[:, :D].T * (1.0 / DEG)
    w2t = W[:, D:].T
    b2 = b.reshape(1, OUT)
    return pl.pallas_call(
        _tc_linear_body,
        grid=(N // _BLK,),
        in_specs=[
            pl.BlockSpec((_BLK, D), lambda i: (i, 0)),
            pl.BlockSpec((_BLK, D), lambda i: (i, 0)),
            pl.BlockSpec((D, OUT), lambda i: (0, 0)),
            pl.BlockSpec((D, OUT), lambda i: (0, 0)),
            pl.BlockSpec((1, OUT), lambda i: (0, 0)),
        ],
        out_specs=pl.BlockSpec((_BLK, OUT), lambda i: (i, 0)),
        out_shape=jax.ShapeDtypeStruct((N, OUT), jnp.float32),
    )(sums, node_feat, w1t, w2t, b2)


# speedup vs baseline: 1.3428x; 1.3428x over previous
import functools

import jax
import jax.numpy as jnp
from jax import lax
from jax.experimental import pallas as pl
from jax.experimental.pallas import tpu as pltpu
from jax.experimental.pallas import tpu_sc as plsc

N = 10000
DEG = 32
D = 128
OUT = 128

_NC = 2
_NS = 16
_NW = _NC * _NS

_CN = 8
_CROWS = _CN * DEG
_NCHUNK = N // _CN
_CPW = -(-_NCHUNK // _NW)

_LANE = 16
_JV = D // _LANE


def _sc_sum_body(h2_hbm, out_hbm, buf0, buf1, sums, sem0, sem1):
    wid = lax.axis_index("s") * _NC + lax.axis_index("c")
    bufs = (buf0, buf1)
    sems = (sem0, sem1)

    def cid_of(i):
        return wid + i * _NW

    def issue(i, slot):
        cid = cid_of(i)

        @pl.when(cid < _NCHUNK)
        def _():
            pltpu.async_copy(
                h2_hbm.at[pl.ds(cid * _CROWS, _CROWS)], bufs[slot], sems[slot]
            )

    def compute(i, slot):
        cid = cid_of(i)

        @pl.when(cid < _NCHUNK)
        def _():
            buf = bufs[slot]
            pltpu.make_async_copy(
                h2_hbm.at[pl.ds(cid * _CROWS, _CROWS)], buf, sems[slot]
            ).wait()

            def node_body(n, carry):
                base = n * DEG
                accs = tuple(buf[base, pl.ds(j * _LANE, _LANE)] for j in range(_JV))
                for dd in range(1, DEG):
                    accs = tuple(
                        accs[j] + buf[base + dd, pl.ds(j * _LANE, _LANE)]
                        for j in range(_JV)
                    )
                for j in range(_JV):
                    sums[n, pl.ds(j * _LANE, _LANE)] = accs[j]
                return carry

            lax.fori_loop(0, _CN, node_body, 0)
            pltpu.sync_copy(sums, out_hbm.at[pl.ds(cid * _CN, _CN)])

    issue(0, 0)

    def loop_body(i2, carry):
        for b in range(2):
            i = i2 * 2 + b
            issue(i + 1, 1 - b)
            compute(i, b)
        return carry

    lax.fori_loop(0, _CPW // 2, loop_body, 0, unroll=False)


@functools.partial(jax.jit, donate_argnums=())
def _sc_mailbox_sum(h2):
    mesh = plsc.VectorSubcoreMesh(
        core_axis_name="c", subcore_axis_name="s",
        num_cores=_NC, num_subcores=_NS,
    )
    return pl.kernel(
        _sc_sum_body,
        out_type=jax.ShapeDtypeStruct((N, D), jnp.float32),
        mesh=mesh,
        scratch_types=[
            pltpu.VMEM((_CROWS, D), jnp.float32),
            pltpu.VMEM((_CROWS, D), jnp.float32),
            pltpu.VMEM((_CN, D), jnp.float32),
            pltpu.SemaphoreType.DMA,
            pltpu.SemaphoreType.DMA,
        ],
    )(h2)


_BLK = 2000


def _tc_linear_body(sums_ref, nf_ref, w1t_ref, w2t_ref, b_ref, out_ref):
    out_ref[...] = (
        jnp.dot(sums_ref[...], w1t_ref[...], preferred_element_type=jnp.float32)
        + jnp.dot(nf_ref[...], w2t_ref[...], preferred_element_type=jnp.float32)
        + b_ref[...]
    )


def kernel(h, node_feat, W, b):
    h2 = h.reshape(N * DEG, D)
    sums = _sc_mailbox_sum(h2)
    w1t = W[:, :D].T * (1.0 / DEG)
    w2t = W[:, D:].T
    b2 = b.reshape(1, OUT)
    return pl.pallas_call(
        _tc_linear_body,
        grid=(N // _BLK,),
        in_specs=[
            pl.BlockSpec((_BLK, D), lambda i: (i, 0)),
            pl.BlockSpec((_BLK, D), lambda i: (i, 0)),
            pl.BlockSpec((D, OUT), lambda i: (0, 0)),
            pl.BlockSpec((D, OUT), lambda i: (0, 0)),
            pl.BlockSpec((1, OUT), lambda i: (0, 0)),
        ],
        out_specs=pl.BlockSpec((_BLK, OUT), lambda i: (i, 0)),
        out_shape=jax.ShapeDtypeStruct((N, OUT), jnp.float32),
    )(sums, node_feat, w1t, w2t, b2)


# speedup vs baseline: 1.3786x; 1.0267x over previous
import functools

import jax
import jax.numpy as jnp
from jax import lax
from jax.experimental import pallas as pl
from jax.experimental.pallas import tpu as pltpu
from jax.experimental.pallas import tpu_sc as plsc

N = 10000
DEG = 32
D = 128
OUT = 128

_NC = 2
_NS = 16
_NW = _NC * _NS

_CN = 8
_CROWS = _CN * DEG
_NCHUNK = N // _CN
_CPW = -(-_NCHUNK // _NW)

_LANE = 16
_JV = D // _LANE


def _sc_sum_body(h2_hbm, out_hbm, buf0, buf1, sums0, sums1, sem0, sem1, osem0, osem1):
    wid = lax.axis_index("s") * _NC + lax.axis_index("c")
    bufs = (buf0, buf1)
    sums_ = (sums0, sums1)
    sems = (sem0, sem1)
    osems = (osem0, osem1)

    def cid_of(i):
        return wid + i * _NW

    def issue(i, slot):
        cid = cid_of(i)

        @pl.when(cid < _NCHUNK)
        def _():
            pltpu.async_copy(
                h2_hbm.at[pl.ds(cid * _CROWS, _CROWS)], bufs[slot], sems[slot]
            )

    def wait_out(i, slot):
        cid = cid_of(i)

        @pl.when((i >= 0) & (cid < _NCHUNK))
        def _():
            pltpu.make_async_copy(
                sums_[slot], out_hbm.at[pl.ds(cid * _CN, _CN)], osems[slot]
            ).wait()

    def compute(i, slot):
        cid = cid_of(i)
        wait_out(i - 2, slot)

        @pl.when(cid < _NCHUNK)
        def _():
            buf = bufs[slot]
            sm = sums_[slot]
            pltpu.make_async_copy(
                h2_hbm.at[pl.ds(cid * _CROWS, _CROWS)], buf, sems[slot]
            ).wait()

            def node_body(n, carry):
                base = n * DEG
                accs = tuple(buf[base, pl.ds(j * _LANE, _LANE)] for j in range(_JV))
                for dd in range(1, DEG):
                    accs = tuple(
                        accs[j] + buf[base + dd, pl.ds(j * _LANE, _LANE)]
                        for j in range(_JV)
                    )
                for j in range(_JV):
                    sm[n, pl.ds(j * _LANE, _LANE)] = accs[j]
                return carry

            lax.fori_loop(0, _CN, node_body, 0)
            pltpu.async_copy(sm, out_hbm.at[pl.ds(cid * _CN, _CN)], osems[slot])

    issue(0, 0)

    def loop_body(i2, carry):
        for b in range(2):
            i = i2 * 2 + b
            issue(i + 1, 1 - b)
            compute(i, b)
        return carry

    lax.fori_loop(0, _CPW // 2, loop_body, 0, unroll=False)
    for i in (_CPW - 2, _CPW - 1):
        wait_out(i, i % 2)


@functools.partial(jax.jit, donate_argnums=())
def _sc_mailbox_sum(h2):
    mesh = plsc.VectorSubcoreMesh(
        core_axis_name="c", subcore_axis_name="s",
        num_cores=_NC, num_subcores=_NS,
    )
    return pl.kernel(
        _sc_sum_body,
        out_type=jax.ShapeDtypeStruct((N, D), jnp.float32),
        mesh=mesh,
        scratch_types=[
            pltpu.VMEM((_CROWS, D), jnp.float32),
            pltpu.VMEM((_CROWS, D), jnp.float32),
            pltpu.VMEM((_CN, D), jnp.float32),
            pltpu.VMEM((_CN, D), jnp.float32),
            pltpu.SemaphoreType.DMA,
            pltpu.SemaphoreType.DMA,
            pltpu.SemaphoreType.DMA,
            pltpu.SemaphoreType.DMA,
        ],
    )(h2)


_BLK = 2000


def _tc_linear_body(sums_ref, nf_ref, w1t_ref, w2t_ref, b_ref, out_ref):
    out_ref[...] = (
        jnp.dot(sums_ref[...], w1t_ref[...], preferred_element_type=jnp.float32)
        + jnp.dot(nf_ref[...], w2t_ref[...], preferred_element_type=jnp.float32)
        + b_ref[...]
    )


def kernel(h, node_feat, W, b):
    h2 = h.reshape(N * DEG, D)
    sums = _sc_mailbox_sum(h2)
    w1t = W[:, :D].T * (1.0 / DEG)
    w2t = W[:, D:].T
    b2 = b.reshape(1, OUT)
    return pl.pallas_call(
        _tc_linear_body,
        grid=(N // _BLK,),
        in_specs=[
            pl.BlockSpec((_BLK, D), lambda i: (i, 0)),
            pl.BlockSpec((_BLK, D), lambda i: (i, 0)),
            pl.BlockSpec((D, OUT), lambda i: (0, 0)),
            pl.BlockSpec((D, OUT), lambda i: (0, 0)),
            pl.BlockSpec((1, OUT), lambda i: (0, 0)),
        ],
        out_specs=pl.BlockSpec((_BLK, OUT), lambda i: (i, 0)),
        out_shape=jax.ShapeDtypeStruct((N, OUT), jnp.float32),
    )(sums, node_feat, w1t, w2t, b2)


# speedup vs baseline: 1.9148x; 1.3889x over previous
import functools

import jax
import jax.numpy as jnp
from jax import lax
from jax.experimental import pallas as pl
from jax.experimental.pallas import tpu as pltpu
from jax.experimental.pallas import tpu_sc as plsc

N = 10000
DEG = 32
D = 128
OUT = 128

_NSC = 4000

_NC = 2
_NS = 16
_NW = _NC * _NS

_CN = 8
_CROWS = _CN * DEG
_NCHUNK = _NSC // _CN
_CPW = -(-_NCHUNK // _NW)

_LANE = 16
_JV = D // _LANE


def _sc_sum_body(h2_hbm, out_hbm, buf0, buf1, sums0, sums1, sem0, sem1, osem0, osem1):
    wid = lax.axis_index("s") * _NC + lax.axis_index("c")
    bufs = (buf0, buf1)
    sums_ = (sums0, sums1)
    sems = (sem0, sem1)
    osems = (osem0, osem1)

    def cid_of(i):
        return wid + i * _NW

    def issue(i, slot):
        cid = cid_of(i)

        @pl.when(cid < _NCHUNK)
        def _():
            pltpu.async_copy(
                h2_hbm.at[pl.ds(cid * _CROWS, _CROWS)], bufs[slot], sems[slot]
            )

    def wait_out(i, slot):
        cid = cid_of(i)

        @pl.when((i >= 0) & (cid < _NCHUNK))
        def _():
            pltpu.make_async_copy(
                sums_[slot], out_hbm.at[pl.ds(cid * _CN, _CN)], osems[slot]
            ).wait()

    def compute(i, slot):
        cid = cid_of(i)
        wait_out(i - 2, slot)

        @pl.when(cid < _NCHUNK)
        def _():
            buf = bufs[slot]
            sm = sums_[slot]
            pltpu.make_async_copy(
                h2_hbm.at[pl.ds(cid * _CROWS, _CROWS)], buf, sems[slot]
            ).wait()

            def node_body(n, carry):
                base = n * DEG
                accs = tuple(buf[base, pl.ds(j * _LANE, _LANE)] for j in range(_JV))
                for dd in range(1, DEG):
                    accs = tuple(
                        accs[j] + buf[base + dd, pl.ds(j * _LANE, _LANE)]
                        for j in range(_JV)
                    )
                for j in range(_JV):
                    sm[n, pl.ds(j * _LANE, _LANE)] = accs[j]
                return carry

            lax.fori_loop(0, _CN, node_body, 0)
            pltpu.async_copy(sm, out_hbm.at[pl.ds(cid * _CN, _CN)], osems[slot])

    issue(0, 0)

    def loop_body(i2, carry):
        for b in range(2):
            i = i2 * 2 + b
            issue(i + 1, 1 - b)
            compute(i, b)
        return carry

    lax.fori_loop(0, _CPW // 2, loop_body, 0, unroll=False)
    for i in (_CPW - 2, _CPW - 1):
        wait_out(i, i % 2)


@functools.partial(jax.jit, donate_argnums=())
def _sc_mailbox_sum(h2):
    mesh = plsc.VectorSubcoreMesh(
        core_axis_name="c", subcore_axis_name="s",
        num_cores=_NC, num_subcores=_NS,
    )
    return pl.kernel(
        _sc_sum_body,
        out_type=jax.ShapeDtypeStruct((_NSC, D), jnp.float32),
        mesh=mesh,
        scratch_types=[
            pltpu.VMEM((_CROWS, D), jnp.float32),
            pltpu.VMEM((_CROWS, D), jnp.float32),
            pltpu.VMEM((_CN, D), jnp.float32),
            pltpu.VMEM((_CN, D), jnp.float32),
            pltpu.SemaphoreType.DMA,
            pltpu.SemaphoreType.DMA,
            pltpu.SemaphoreType.DMA,
            pltpu.SemaphoreType.DMA,
        ],
    )(h2)


_BLK_A = 1000
_BLK_B = 1000
_OFF_A = _NSC // _BLK_A


def _tc_fused_body(h_ref, nf_ref, w1t_ref, w2t_ref, b_ref, out_ref):
    hm = jnp.mean(h_ref[...], axis=1)
    out_ref[...] = (
        jnp.dot(hm, w1t_ref[...], preferred_element_type=jnp.float32)
        + jnp.dot(nf_ref[...], w2t_ref[...], preferred_element_type=jnp.float32)
        + b_ref[...]
    )


def _tc_linear_body(sums_ref, nf_ref, w1t_ref, w2t_ref, b_ref, out_ref):
    out_ref[...] = (
        jnp.dot(sums_ref[...], w1t_ref[...], preferred_element_type=jnp.float32)
        + jnp.dot(nf_ref[...], w2t_ref[...], preferred_element_type=jnp.float32)
        + b_ref[...]
    )


def kernel(h, node_feat, W, b):
    h2 = h.reshape(N * DEG, D)
    sums = _sc_mailbox_sum(h2)

    w1t = W[:, :D].T
    w1t_s = w1t * (1.0 / DEG)
    w2t = W[:, D:].T
    b2 = b.reshape(1, OUT)

    out_a = pl.pallas_call(
        _tc_fused_body,
        grid=((N - _NSC) // _BLK_A,),
        in_specs=[
            pl.BlockSpec((_BLK_A, DEG, D), lambda i: (i + _OFF_A, 0, 0)),
            pl.BlockSpec((_BLK_A, D), lambda i: (i + _OFF_A, 0)),
            pl.BlockSpec((D, OUT), lambda i: (0, 0)),
            pl.BlockSpec((D, OUT), lambda i: (0, 0)),
            pl.BlockSpec((1, OUT), lambda i: (0, 0)),
        ],
        out_specs=pl.BlockSpec((_BLK_A, OUT), lambda i: (i, 0)),
        out_shape=jax.ShapeDtypeStruct((N - _NSC, OUT), jnp.float32),
    )(h, node_feat, w1t, w2t, b2)

    out_b = pl.pallas_call(
        _tc_linear_body,
        grid=(_NSC // _BLK_B,),
        in_specs=[
            pl.BlockSpec((_BLK_B, D), lambda i: (i, 0)),
            pl.BlockSpec((_BLK_B, D), lambda i: (i, 0)),
            pl.BlockSpec((D, OUT), lambda i: (0, 0)),
            pl.BlockSpec((D, OUT), lambda i: (0, 0)),
            pl.BlockSpec((1, OUT), lambda i: (0, 0)),
        ],
        out_specs=pl.BlockSpec((_BLK_B, OUT), lambda i: (i, 0)),
        out_shape=jax.ShapeDtypeStruct((_NSC, OUT), jnp.float32),
    )(sums, node_feat, w1t_s, w2t, b2)

    return jnp.concatenate([out_b, out_a], axis=0)


# speedup vs baseline: 2.0520x; 1.0716x over previous
import functools

import jax
import jax.numpy as jnp
from jax import lax
from jax.experimental import pallas as pl
from jax.experimental.pallas import tpu as pltpu
from jax.experimental.pallas import tpu_sc as plsc

N = 10000
DEG = 32
D = 128
OUT = 128

_NSC = 4000

_NC = 2
_NS = 16
_NW = _NC * _NS

_CN = 8
_CROWS = _CN * DEG
_NCHUNK = _NSC // _CN
_CPW = -(-_NCHUNK // _NW)

_LANE = 16
_JV = D // _LANE


def _sc_sum_body(h2_hbm, out_hbm, buf0, buf1, sums0, sums1, sem0, sem1, osem0, osem1):
    wid = lax.axis_index("s") * _NC + lax.axis_index("c")
    bufs = (buf0, buf1)
    sums_ = (sums0, sums1)
    sems = (sem0, sem1)
    osems = (osem0, osem1)

    def cid_of(i):
        return wid + i * _NW

    def issue(i, slot):
        cid = cid_of(i)

        @pl.when(cid < _NCHUNK)
        def _():
            pltpu.async_copy(
                h2_hbm.at[pl.ds(cid * _CROWS, _CROWS)], bufs[slot], sems[slot]
            )

    def wait_out(i, slot):
        cid = cid_of(i)

        @pl.when((i >= 0) & (cid < _NCHUNK))
        def _():
            pltpu.make_async_copy(
                sums_[slot], out_hbm.at[pl.ds(cid * _CN, _CN)], osems[slot]
            ).wait()

    def compute(i, slot):
        cid = cid_of(i)
        wait_out(i - 2, slot)

        @pl.when(cid < _NCHUNK)
        def _():
            buf = bufs[slot]
            sm = sums_[slot]
            pltpu.make_async_copy(
                h2_hbm.at[pl.ds(cid * _CROWS, _CROWS)], buf, sems[slot]
            ).wait()

            def node_body(n, carry):
                base = n * DEG
                accs = tuple(buf[base, pl.ds(j * _LANE, _LANE)] for j in range(_JV))
                for dd in range(1, DEG):
                    accs = tuple(
                        accs[j] + buf[base + dd, pl.ds(j * _LANE, _LANE)]
                        for j in range(_JV)
                    )
                for j in range(_JV):
                    sm[n, pl.ds(j * _LANE, _LANE)] = accs[j]
                return carry

            lax.fori_loop(0, _CN, node_body, 0)
            pltpu.async_copy(sm, out_hbm.at[pl.ds(cid * _CN, _CN)], osems[slot])

    issue(0, 0)

    def loop_body(i2, carry):
        for b in range(2):
            i = i2 * 2 + b
            issue(i + 1, 1 - b)
            compute(i, b)
        return carry

    lax.fori_loop(0, _CPW // 2, loop_body, 0, unroll=False)
    for i in (_CPW - 2, _CPW - 1):
        wait_out(i, i % 2)


@functools.partial(jax.jit, donate_argnums=())
def _sc_mailbox_sum(h2):
    mesh = plsc.VectorSubcoreMesh(
        core_axis_name="c", subcore_axis_name="s",
        num_cores=_NC, num_subcores=_NS,
    )
    return pl.kernel(
        _sc_sum_body,
        out_type=jax.ShapeDtypeStruct((_NSC, D), jnp.float32),
        mesh=mesh,
        scratch_types=[
            pltpu.VMEM((_CROWS, D), jnp.float32),
            pltpu.VMEM((_CROWS, D), jnp.float32),
            pltpu.VMEM((_CN, D), jnp.float32),
            pltpu.VMEM((_CN, D), jnp.float32),
            pltpu.SemaphoreType.DMA,
            pltpu.SemaphoreType.DMA,
            pltpu.SemaphoreType.DMA,
            pltpu.SemaphoreType.DMA,
        ],
    )(h2)


_BLK_A = 1000
_BLK_B = 1000
_OFF_A = _NSC // _BLK_A


def _tc_fused_body(h_ref, nf_ref, w1t_ref, w2t_ref, b_ref, out_ref):
    hm = jnp.mean(h_ref[...], axis=1)
    out_ref[...] = (
        jnp.dot(hm, w1t_ref[...], preferred_element_type=jnp.float32)
        + jnp.dot(nf_ref[...], w2t_ref[...], preferred_element_type=jnp.float32)
        + b_ref[...]
    )


def _tc_linear_body(sums_ref, nf_ref, w1t_ref, w2t_ref, b_ref, _partial_ref, out_ref):
    out_ref[...] = (
        jnp.dot(sums_ref[...], w1t_ref[...], preferred_element_type=jnp.float32)
        + jnp.dot(nf_ref[...], w2t_ref[...], preferred_element_type=jnp.float32)
        + b_ref[...]
    )


def kernel(h, node_feat, W, b):
    h2 = h.reshape(N * DEG, D)
    sums = _sc_mailbox_sum(h2)

    w1t = W[:, :D].T
    w1t_s = w1t * (1.0 / DEG)
    w2t = W[:, D:].T
    b2 = b.reshape(1, OUT)

    out_a = pl.pallas_call(
        _tc_fused_body,
        grid=((N - _NSC) // _BLK_A,),
        in_specs=[
            pl.BlockSpec((_BLK_A, DEG, D), lambda i: (i + _OFF_A, 0, 0)),
            pl.BlockSpec((_BLK_A, D), lambda i: (i + _OFF_A, 0)),
            pl.BlockSpec((D, OUT), lambda i: (0, 0)),
            pl.BlockSpec((D, OUT), lambda i: (0, 0)),
            pl.BlockSpec((1, OUT), lambda i: (0, 0)),
        ],
        out_specs=pl.BlockSpec((_BLK_A, OUT), lambda i: (i + _OFF_A, 0)),
        out_shape=jax.ShapeDtypeStruct((N, OUT), jnp.float32),
    )(h, node_feat, w1t, w2t, b2)

    return pl.pallas_call(
        _tc_linear_body,
        grid=(_NSC // _BLK_B,),
        in_specs=[
            pl.BlockSpec((_BLK_B, D), lambda i: (i, 0)),
            pl.BlockSpec((_BLK_B, D), lambda i: (i, 0)),
            pl.BlockSpec((D, OUT), lambda i: (0, 0)),
            pl.BlockSpec((D, OUT), lambda i: (0, 0)),
            pl.BlockSpec((1, OUT), lambda i: (0, 0)),
            pl.BlockSpec(memory_space=pl.ANY),
        ],
        out_specs=pl.BlockSpec((_BLK_B, OUT), lambda i: (i, 0)),
        out_shape=jax.ShapeDtypeStruct((N, OUT), jnp.float32),
        input_output_aliases={5: 0},
    )(sums, node_feat, w1t_s, w2t, b2, out_a)


# speedup vs baseline: 2.0698x; 1.0087x over previous
import functools

import jax
import jax.numpy as jnp
from jax import lax
from jax.experimental import pallas as pl
from jax.experimental.pallas import tpu as pltpu
from jax.experimental.pallas import tpu_sc as plsc

N = 10000
DEG = 32
D = 128
OUT = 128

_NSC = 2000

_NC = 2
_NS = 16
_NW = _NC * _NS

_CN = 8
_CROWS = _CN * DEG
_NCHUNK = _NSC // _CN
_CPW = -(-_NCHUNK // _NW)

_LANE = 16
_JV = D // _LANE


def _sc_sum_body(h2_hbm, out_hbm, buf0, buf1, sums0, sums1, sem0, sem1, osem0, osem1):
    wid = lax.axis_index("s") * _NC + lax.axis_index("c")
    bufs = (buf0, buf1)
    sums_ = (sums0, sums1)
    sems = (sem0, sem1)
    osems = (osem0, osem1)

    def cid_of(i):
        return wid + i * _NW

    def issue(i, slot):
        cid = cid_of(i)

        @pl.when(cid < _NCHUNK)
        def _():
            pltpu.async_copy(
                h2_hbm.at[pl.ds(cid * _CROWS, _CROWS)], bufs[slot], sems[slot]
            )

    def wait_out(i, slot):
        cid = cid_of(i)

        @pl.when((i >= 0) & (cid < _NCHUNK))
        def _():
            pltpu.make_async_copy(
                sums_[slot], out_hbm.at[pl.ds(cid * _CN, _CN)], osems[slot]
            ).wait()

    def compute(i, slot):
        cid = cid_of(i)
        wait_out(i - 2, slot)

        @pl.when(cid < _NCHUNK)
        def _():
            buf = bufs[slot]
            sm = sums_[slot]
            pltpu.make_async_copy(
                h2_hbm.at[pl.ds(cid * _CROWS, _CROWS)], buf, sems[slot]
            ).wait()

            def node_body(n, carry):
                base = n * DEG
                accs = tuple(buf[base, pl.ds(j * _LANE, _LANE)] for j in range(_JV))
                for dd in range(1, DEG):
                    accs = tuple(
                        accs[j] + buf[base + dd, pl.ds(j * _LANE, _LANE)]
                        for j in range(_JV)
                    )
                for j in range(_JV):
                    sm[n, pl.ds(j * _LANE, _LANE)] = accs[j] * (1.0 / DEG)
                return carry

            lax.fori_loop(0, _CN, node_body, 0)
            pltpu.async_copy(sm, out_hbm.at[pl.ds(cid * _CN, _CN)], osems[slot])

    issue(0, 0)

    def loop_body(i2, carry):
        for b in range(2):
            i = i2 * 2 + b
            issue(i + 1, 1 - b)
            compute(i, b)
        return carry

    lax.fori_loop(0, _CPW // 2, loop_body, 0, unroll=False)
    for i in (_CPW - 2, _CPW - 1):
        wait_out(i, i % 2)


@functools.partial(jax.jit, donate_argnums=())
def _sc_mailbox_sum(h2):
    mesh = plsc.VectorSubcoreMesh(
        core_axis_name="c", subcore_axis_name="s",
        num_cores=_NC, num_subcores=_NS,
    )
    return pl.kernel(
        _sc_sum_body,
        out_type=jax.ShapeDtypeStruct((_NSC, D), jnp.float32),
        mesh=mesh,
        scratch_types=[
            pltpu.VMEM((_CROWS, D), jnp.float32),
            pltpu.VMEM((_CROWS, D), jnp.float32),
            pltpu.VMEM((_CN, D), jnp.float32),
            pltpu.VMEM((_CN, D), jnp.float32),
            pltpu.SemaphoreType.DMA,
            pltpu.SemaphoreType.DMA,
            pltpu.SemaphoreType.DMA,
            pltpu.SemaphoreType.DMA,
        ],
    )(h2)


_BLK_A = 1000
_BLK_B = 1000
_OFF_A = _NSC // _BLK_A


def _tc_fused_body(h_ref, nf_ref, w1t_ref, w2t_ref, b_ref, out_ref):
    hm = jnp.mean(h_ref[...], axis=1)
    hterm = jnp.dot(hm, w1t_ref[...], preferred_element_type=jnp.float32)
    base = (
        jnp.dot(nf_ref[...], w2t_ref[...], preferred_element_type=jnp.float32)
        + b_ref[...]
    )
    keep = (pl.program_id(0) >= _OFF_A).astype(jnp.float32)
    out_ref[...] = base + keep * hterm


def _tc_acc_body(means_ref, w1t_ref, _partial_ref, out_ref):
    out_ref[...] = out_ref[...] + jnp.dot(
        means_ref[...], w1t_ref[...], preferred_element_type=jnp.float32
    )


def kernel(h, node_feat, W, b):
    h2 = h.reshape(N * DEG, D)
    means = _sc_mailbox_sum(h2)

    w1t = W[:, :D].T
    w2t = W[:, D:].T
    b2 = b.reshape(1, OUT)

    out_a = pl.pallas_call(
        _tc_fused_body,
        grid=(N // _BLK_A,),
        in_specs=[
            pl.BlockSpec((_BLK_A, DEG, D), lambda i: (jnp.maximum(i, _OFF_A), 0, 0)),
            pl.BlockSpec((_BLK_A, D), lambda i: (i, 0)),
            pl.BlockSpec((D, OUT), lambda i: (0, 0)),
            pl.BlockSpec((D, OUT), lambda i: (0, 0)),
            pl.BlockSpec((1, OUT), lambda i: (0, 0)),
        ],
        out_specs=pl.BlockSpec((_BLK_A, OUT), lambda i: (i, 0)),
        out_shape=jax.ShapeDtypeStruct((N, OUT), jnp.float32),
    )(h, node_feat, w1t, w2t, b2)

    return pl.pallas_call(
        _tc_acc_body,
        grid=(_NSC // _BLK_B,),
        in_specs=[
            pl.BlockSpec((_BLK_B, D), lambda i: (i, 0)),
            pl.BlockSpec((D, OUT), lambda i: (0, 0)),
            pl.BlockSpec(memory_space=pl.ANY),
        ],
        out_specs=pl.BlockSpec((_BLK_B, OUT), lambda i: (i, 0)),
        out_shape=jax.ShapeDtypeStruct((N, OUT), jnp.float32),
        input_output_aliases={2: 0},
    )(means, w1t, out_a)
